# Initial kernel scaffold; baseline (speedup 1.0000x reference)
import jax
import jax.numpy as jnp
from jax.experimental import pallas as pl


def kernel(input, edge, edge_embed, edge_list_nhop, edge_embed_nhop, a, a__2, a_2, W_mlp, b_mlp):
    raise NotImplementedError("write your pallas kernel here")


# speedup vs baseline: 1.7704x; 1.7704x over previous
import jax
import jax.numpy as jnp
from jax import lax
from jax.experimental import pallas as pl
from jax.experimental.pallas import tpu as pltpu
from jax.experimental.pallas import tpu_sc as plsc

N_NODES = 10000
N_PAD = 10240
D = 128
NRELA = 16
E_TOTAL = 320000
E_PAD = 327680
NC = 2
NS = 16
K = 80
EPT = E_PAD // (NC * NS)
NCHUNK = EPT // K
STRIPE = N_PAD // NS
BIG = 1.0e9


def _prep_nodes_body(x_ref, aT_ref, m_ref, p1_ref, p2_ref, q_ref):
    x = x_ref[...]
    p1 = jnp.dot(x, aT_ref[0:D, :], preferred_element_type=jnp.float32)
    p2 = jnp.dot(x, aT_ref[D:2 * D, :], preferred_element_type=jnp.float32)
    p1_ref[...] = p1
    p2_ref[...] = p2
    m = m_ref[...]
    q_ref[...] = (jnp.dot(p1, m[:, 0:4], preferred_element_type=jnp.float32)
                  + jnp.dot(p2, m[:, 4:8], preferred_element_type=jnp.float32))


def _prep_edges_body(ee_ref, a3T_ref, cwb_ref, eex_ref):
    pid = pl.program_id(0)
    blk = ee_ref.shape[0]
    ee = ee_ref[...]
    a3T = a3T_ref[...]
    c = cwb_ref[0, :]
    w = cwb_ref[1, :]
    b = cwb_ref[2, 0]
    cv3 = jnp.dot(a3T, c, preferred_element_type=jnp.float32)
    dv3 = jnp.dot(a3T, w, preferred_element_type=jnp.float32)
    gi = pid * blk + lax.broadcasted_iota(jnp.int32, (blk, 1), 0)
    pad = gi >= E_TOTAL
    qe = jnp.dot(ee, cv3[:, None], preferred_element_type=jnp.float32)
    re = jnp.dot(ee, dv3[:, None], preferred_element_type=jnp.float32) + b
    qe = jnp.where(pad, BIG, qe)
    re = jnp.where(pad, BIG, re)
    ones = jnp.where(pad, 0.0, 1.0).astype(jnp.float32)
    eex_ref[...] = jnp.concatenate(
        [ones, ee, qe, re, jnp.zeros((blk, 13), jnp.float32)], axis=1)


def _post_body(zr0_ref, zr1_ref, h20_ref, h21_ref, p1_ref, a3T_ref, out_ref):
    zr = zr0_ref[...] + zr1_ref[...]
    rs = zr[:, 0:1]
    z = zr[:, 1:1 + NRELA]
    num = (h20_ref[...] + h21_ref[...] + p1_ref[...] * rs
           + jnp.dot(z, a3T_ref[...], preferred_element_type=jnp.float32))
    den = jnp.where(rs == 0.0, 1e-12, rs)
    h = num / den
    out_ref[...] = jnp.where(h > 0.0, h, jnp.exp(h) - 1.0)


def _sc_body(edata_hbm, eex_hbm, qa_hbm, qb_hbm, p2_hbm,
             zr_out, h2_out,
             edata_v, src_v, dst_v, qa_v, qb_v, eex_v, prow_v,
             zr_sh, h2_sh, sem):
    cid = lax.axis_index("c")
    sid = lax.axis_index("s")
    ebase = (cid * NS + sid) * EPT
    nb = sid * STRIPE

    iota16 = jnp.arange(16, dtype=jnp.int32)
    zeros16 = jnp.zeros((16,), jnp.int32)
    zero16f = jnp.zeros((16,), jnp.float32)

    def _zrow(i, carry):
        for r in range(D // 16):
            prow_v[i, pl.ds(16 * r, 16)] = zero16f
        eex_v[i, pl.ds(0, 16)] = zero16f
        eex_v[i, pl.ds(16, 16)] = zero16f
        return carry

    lax.fori_loop(0, K, _zrow, 0)
    for t in range(STRIPE // K):
        pltpu.sync_copy(prow_v, h2_sh.at[pl.ds(nb + K * t, K)])
        pltpu.sync_copy(eex_v, zr_sh.at[pl.ds(nb + K * t, K)])
    plsc.subcore_barrier()

    inv_e = -1.0 / float(E_TOTAL)

    def _chunk(i, carry):
        base = ebase + i * K
        pltpu.sync_copy(edata_hbm.at[pl.ds(base, K)], edata_v)
        pltpu.sync_copy(eex_hbm.at[pl.ds(base, K)], eex_v)
        for g in range(K // 16):
            rows = 16 * g + iota16
            sl = pl.ds(16 * g, 16)
            src_v[sl] = plsc.load_gather(edata_v, [rows, zeros16])
            dst_v[sl] = plsc.load_gather(edata_v, [rows, zeros16 + 1])
        d1 = pltpu.async_copy(qa_hbm.at[src_v], qa_v, sem)
        d2 = pltpu.async_copy(qb_hbm.at[dst_v], qb_v, sem)
        d3 = pltpu.async_copy(p2_hbm.at[dst_v], prow_v, sem)
        d1.wait()
        d2.wait()
        d3.wait()
        for g in range(K // 16):
            rows = 16 * g + iota16
            q1g = plsc.load_gather(qa_v, [rows, zeros16])
            r1g = plsc.load_gather(qa_v, [rows, zeros16 + 1])
            q2g = plsc.load_gather(qb_v, [rows, zeros16])
            r2g = plsc.load_gather(qb_v, [rows, zeros16 + 1])
            qeg = plsc.load_gather(eex_v, [rows, zeros16 + 17])
            reg = plsc.load_gather(eex_v, [rows, zeros16 + 18])
            s1 = q1g + q2g + qeg
            s2 = r1g + r2g + reg
            u = jnp.exp(-2.0 * jnp.abs(s2))
            th = jnp.sign(s2) * (1.0 - u) / (1.0 + u)
            lr = jnp.where(s1 > 0.0, s1, 0.2 * s1)
            ew = jnp.exp(lr * th * inv_e)
            for j in range(16):
                cs = lax.gather(
                    ew, jnp.full((16, 1), j, jnp.int32),
                    dimension_numbers=lax.GatherDimensionNumbers(
                        offset_dims=(), collapsed_slice_dims=(0,),
                        start_index_map=(0,)),
                    slice_sizes=(1,),
                    mode=lax.GatherScatterMode.PROMISE_IN_BOUNDS)
                row = 16 * g + j
                for r in range(D // 16):
                    prow_v[row, pl.ds(16 * r, 16)] = (
                        prow_v[row, pl.ds(16 * r, 16)] * cs)
                eex_v[row, pl.ds(0, 16)] = eex_v[row, pl.ds(0, 16)] * cs
                eex_v[row, pl.ds(16, 16)] = eex_v[row, pl.ds(16, 16)] * cs
        pltpu.sync_copy(eex_v, zr_sh.at[src_v], add=True)
        pltpu.sync_copy(prow_v, h2_sh.at[src_v], add=True)
        return carry

    lax.fori_loop(0, NCHUNK, _chunk, 0)
    plsc.subcore_barrier()

    ob = cid * N_PAD + nb
    pltpu.sync_copy(zr_sh.at[pl.ds(nb, STRIPE)], zr_out.at[pl.ds(ob, STRIPE)])
    pltpu.sync_copy(h2_sh.at[pl.ds(nb, STRIPE)], h2_out.at[pl.ds(ob, STRIPE)])


def _sc_call(edata, eex, qa, qb, p2):
    f = pl.kernel(
        _sc_body,
        out_type=[jax.ShapeDtypeStruct((NC * N_PAD, 32), jnp.float32),
                  jax.ShapeDtypeStruct((NC * N_PAD, D), jnp.float32)],
        mesh=plsc.VectorSubcoreMesh(core_axis_name="c", subcore_axis_name="s"),
        compiler_params=pltpu.CompilerParams(needs_layout_passes=False, use_tc_tiling_on_sc=False),
        scratch_types=[
            pltpu.VMEM((K, 2), jnp.int32),
            pltpu.VMEM((K,), jnp.int32),
            pltpu.VMEM((K,), jnp.int32),
            pltpu.VMEM((K, 2), jnp.float32),
            pltpu.VMEM((K, 2), jnp.float32),
            pltpu.VMEM((K, 32), jnp.float32),
            pltpu.VMEM((K, D), jnp.float32),
            pltpu.VMEM_SHARED((N_PAD, 32), jnp.float32),
            pltpu.VMEM_SHARED((N_PAD, D), jnp.float32),
            pltpu.SemaphoreType.DMA,
        ],
    )
    return f(edata, eex, qa, qb, p2)


def kernel(input, edge, edge_embed, edge_list_nhop, edge_embed_nhop,
           a, a__2, a_2, W_mlp, b_mlp):
    x = jnp.pad(input, ((0, N_PAD - N_NODES), (0, 0)))
    aT = a.T
    c = a_2[0]
    w = W_mlp[0]
    zc = jnp.zeros_like(c)
    m8 = jnp.stack([c, w, zc, zc, zc, zc, c, w], axis=1)
    a3T = aT[2 * D:2 * D + NRELA, :]
    cwb = jnp.concatenate(
        [a_2, W_mlp, jnp.broadcast_to(b_mlp[0], (1, D)),
         jnp.zeros((5, D), jnp.float32)], axis=0)

    src = jnp.concatenate([edge[0], edge_list_nhop[0]])
    dst = jnp.concatenate([edge[1], edge_list_nhop[1]])
    src = jnp.pad(src, (0, E_PAD - E_TOTAL))
    dst = jnp.pad(dst, (0, E_PAD - E_TOTAL))
    edata = jnp.stack([src, dst], axis=1)
    ee = jnp.concatenate([edge_embed, edge_embed_nhop], axis=0)
    ee = jnp.pad(ee, ((0, E_PAD - E_TOTAL), (0, 0)))

    nblk = 512
    p1, p2, q = pl.pallas_call(
        _prep_nodes_body,
        grid=(N_PAD // nblk,),
        in_specs=[
            pl.BlockSpec((nblk, D), lambda i: (i, 0)),
            pl.BlockSpec((2 * D + NRELA, D), lambda i: (0, 0)),
            pl.BlockSpec((D, 8), lambda i: (0, 0)),
        ],
        out_specs=[
            pl.BlockSpec((nblk, D), lambda i: (i, 0)),
            pl.BlockSpec((nblk, D), lambda i: (i, 0)),
            pl.BlockSpec((nblk, 4), lambda i: (i, 0)),
        ],
        out_shape=[
            jax.ShapeDtypeStruct((N_PAD, D), jnp.float32),
            jax.ShapeDtypeStruct((N_PAD, D), jnp.float32),
            jax.ShapeDtypeStruct((N_PAD, 4), jnp.float32),
        ],
    )(x, aT, m8)
    qa = q[:, 0:2]
    qb = q[:, 2:4]

    eblk = 2048
    eex = pl.pallas_call(
        _prep_edges_body,
        grid=(E_PAD // eblk,),
        in_specs=[
            pl.BlockSpec((eblk, NRELA), lambda i: (i, 0)),
            pl.BlockSpec((NRELA, D), lambda i: (0, 0)),
            pl.BlockSpec((8, D), lambda i: (0, 0)),
        ],
        out_specs=pl.BlockSpec((eblk, 32), lambda i: (i, 0)),
        out_shape=jax.ShapeDtypeStruct((E_PAD, 32), jnp.float32),
    )(ee, a3T, cwb)

    zr, h2 = _sc_call(edata, eex, qa, qb, p2)

    pblk = 512
    h = pl.pallas_call(
        _post_body,
        grid=(N_PAD // pblk,),
        in_specs=[
            pl.BlockSpec((pblk, 32), lambda i: (i, 0)),
            pl.BlockSpec((pblk, 32), lambda i: (i, 0)),
            pl.BlockSpec((pblk, D), lambda i: (i, 0)),
            pl.BlockSpec((pblk, D), lambda i: (i, 0)),
            pl.BlockSpec((pblk, D), lambda i: (i, 0)),
            pl.BlockSpec((NRELA, D), lambda i: (0, 0)),
        ],
        out_specs=pl.BlockSpec((pblk, D), lambda i: (i, 0)),
        out_shape=jax.ShapeDtypeStruct((N_PAD, D), jnp.float32),
    )(zr[:N_PAD], zr[N_PAD:], h2[:N_PAD], h2[N_PAD:], p1, a3T)

    return h[:N_NODES]


# speedup vs baseline: 4.1100x; 2.3215x over previous
import jax
import jax.numpy as jnp
from jax import lax
from jax.experimental import pallas as pl
from jax.experimental.pallas import tpu as pltpu
from jax.experimental.pallas import tpu_sc as plsc

N_NODES = 10000
N_PAD = 10240
D = 128
NRELA = 16
E1 = 256000
E2 = 64000
E_TOTAL = E1 + E2
NC = 2
NS = 16
K = 80
EPT = E_TOTAL // (NC * NS)
NCHUNK = EPT // K
STRIPE = N_PAD // NS


def _prep_nodes_body(x_ref, aT_ref, m_ref, b_ref,
                     p1_ref, p2_ref, qa_ref, qb_ref, cvd_ref):
    x = x_ref[...]
    p1 = jnp.dot(x, aT_ref[0:D, :], preferred_element_type=jnp.float32)
    p2 = jnp.dot(x, aT_ref[D:2 * D, :], preferred_element_type=jnp.float32)
    p1_ref[...] = p1
    p2_ref[...] = p2
    m = m_ref[...]
    qa_ref[...] = jnp.dot(p1, m, preferred_element_type=jnp.float32)
    qb_ref[...] = (jnp.dot(p2, m, preferred_element_type=jnp.float32)
                   + b_ref[0:1, :])
    cvd_ref[...] = lax.dot_general(
        m, aT_ref[2 * D:2 * D + NRELA, :],
        dimension_numbers=(((0,), (1,)), ((), ())),
        preferred_element_type=jnp.float32)


def _post_body(zr0_ref, zr1_ref, h20_ref, h21_ref, p1_ref, a3T_ref, out_ref):
    zr = zr0_ref[...] + zr1_ref[...]
    z = zr[:, 0:NRELA]
    rs = zr[:, NRELA:NRELA + 1]
    num = (h20_ref[...] + h21_ref[...] + p1_ref[...] * rs
           + jnp.dot(z, a3T_ref[...], preferred_element_type=jnp.float32))
    den = jnp.where(rs == 0.0, 1e-12, rs)
    h = num / den
    out_ref[...] = jnp.where(h > 0.0, h, jnp.exp(h) - 1.0)


def _lane_bcast(v, j):
    return lax.gather(
        v, jnp.full((16, 1), j, jnp.int32),
        dimension_numbers=lax.GatherDimensionNumbers(
            offset_dims=(), collapsed_slice_dims=(0,), start_index_map=(0,)),
        slice_sizes=(1,),
        mode=lax.GatherScatterMode.PROMISE_IN_BOUNDS)


def _sc_body(src_hbm, dst_hbm, ee1_hbm, ee2_hbm, qa_hbm, qb_hbm, p2_hbm,
             cvd_hbm,
             zr_out, h2_out,
             src_v, dst_v, ee_v, pay_v, prow_v, qa_v, qb_v, cvd_v,
             zr_sh, h2_sh, sem):
    cid = lax.axis_index("c")
    sid = lax.axis_index("s")
    ebase = (cid * NS + sid) * EPT
    nb = sid * STRIPE

    iota16 = jnp.arange(16, dtype=jnp.int32)
    zeros16 = jnp.zeros((16,), jnp.int32)
    zero16f = jnp.zeros((16,), jnp.float32)
    onehot0 = jnp.where(iota16 == 0, 1.0, 0.0).astype(jnp.float32)

    pltpu.sync_copy(cvd_hbm, cvd_v)

    def _zrow(i, carry):
        for r in range(D // 16):
            prow_v[i, pl.ds(16 * r, 16)] = zero16f
        pay_v[i, pl.ds(0, 16)] = zero16f
        pay_v[i, pl.ds(16, 16)] = zero16f
        return carry

    lax.fori_loop(0, K, _zrow, 0)
    for t in range(STRIPE // K):
        pltpu.sync_copy(prow_v, h2_sh.at[pl.ds(nb + K * t, K)])
        pltpu.sync_copy(pay_v, zr_sh.at[pl.ds(nb + K * t, K)])
    plsc.subcore_barrier()

    cv3 = cvd_v[0, :]
    dv3 = cvd_v[1, :]
    cvb = [_lane_bcast(cv3, c) for c in range(NRELA)]
    dvb = [_lane_bcast(dv3, c) for c in range(NRELA)]

    inv_e = -1.0 / float(E_TOTAL)

    def _chunk(i, carry):
        base = ebase + i * K
        pltpu.sync_copy(src_hbm.at[pl.ds(base, K)], src_v)
        pltpu.sync_copy(dst_hbm.at[pl.ds(base, K)], dst_v)

        @pl.when(base < E1)
        def _():
            pltpu.sync_copy(ee1_hbm.at[pl.ds(base, K)], ee_v)

        @pl.when(base >= E1)
        def _():
            pltpu.sync_copy(ee2_hbm.at[pl.ds(base - E1, K)], ee_v)

        d1 = pltpu.async_copy(qa_hbm.at[src_v], qa_v, sem)
        d2 = pltpu.async_copy(qb_hbm.at[dst_v], qb_v, sem)
        d3 = pltpu.async_copy(p2_hbm.at[dst_v], prow_v, sem)
        d1.wait()
        d2.wait()
        d3.wait()

        for g in range(K // 16):
            rows = 16 * g + iota16
            q1g = plsc.load_gather(qa_v, [rows, zeros16])
            r1g = plsc.load_gather(qa_v, [rows, zeros16 + 1])
            q2g = plsc.load_gather(qb_v, [rows, zeros16])
            r2g = plsc.load_gather(qb_v, [rows, zeros16 + 1])
            col = plsc.load_gather(ee_v, [rows, zeros16])
            qe = col * cvb[0]
            re = col * dvb[0]
            for c in range(1, NRELA):
                col = plsc.load_gather(ee_v, [rows, zeros16 + c])
                qe = qe + col * cvb[c]
                re = re + col * dvb[c]
            s1 = q1g + q2g + qe
            s2 = r1g + r2g + re
            u = jnp.exp(-2.0 * jnp.abs(s2))
            th = jnp.sign(s2) * (1.0 - u) / (1.0 + u)
            lr = jnp.where(s1 > 0.0, s1, 0.2 * s1)
            ew = jnp.exp(lr * th * inv_e)
            for j in range(16):
                cs = _lane_bcast(ew, j)
                row = 16 * g + j
                for r in range(D // 16):
                    prow_v[row, pl.ds(16 * r, 16)] = (
                        prow_v[row, pl.ds(16 * r, 16)] * cs)
                pay_v[row, pl.ds(0, 16)] = ee_v[row, :] * cs
                pay_v[row, pl.ds(16, 16)] = cs * onehot0
        pltpu.sync_copy(pay_v, zr_sh.at[src_v], add=True)
        pltpu.sync_copy(prow_v, h2_sh.at[src_v], add=True)
        return carry

    lax.fori_loop(0, NCHUNK, _chunk, 0)
    plsc.subcore_barrier()

    ob = cid * N_PAD + nb
    pltpu.sync_copy(zr_sh.at[pl.ds(nb, STRIPE)], zr_out.at[pl.ds(ob, STRIPE)])
    pltpu.sync_copy(h2_sh.at[pl.ds(nb, STRIPE)], h2_out.at[pl.ds(ob, STRIPE)])


def _sc_call(src, dst, ee1, ee2, qa, qb, p2, cvd):
    f = pl.kernel(
        _sc_body,
        out_type=[jax.ShapeDtypeStruct((NC * N_PAD, 32), jnp.float32),
                  jax.ShapeDtypeStruct((NC * N_PAD, D), jnp.float32)],
        mesh=plsc.VectorSubcoreMesh(core_axis_name="c", subcore_axis_name="s"),
        compiler_params=pltpu.CompilerParams(
            needs_layout_passes=False, use_tc_tiling_on_sc=False),
        scratch_types=[
            pltpu.VMEM((K,), jnp.int32),
            pltpu.VMEM((K,), jnp.int32),
            pltpu.VMEM((K, NRELA), jnp.float32),
            pltpu.VMEM((K, 32), jnp.float32),
            pltpu.VMEM((K, D), jnp.float32),
            pltpu.VMEM((K, 2), jnp.float32),
            pltpu.VMEM((K, 2), jnp.float32),
            pltpu.VMEM((2, NRELA), jnp.float32),
            pltpu.VMEM_SHARED((N_PAD, 32), jnp.float32),
            pltpu.VMEM_SHARED((N_PAD, D), jnp.float32),
            pltpu.SemaphoreType.DMA,
        ],
    )
    return f(src, dst, ee1, ee2, qa, qb, p2, cvd)


def kernel(input, edge, edge_embed, edge_list_nhop, edge_embed_nhop,
           a, a__2, a_2, W_mlp, b_mlp):
    x = jnp.pad(input, ((0, N_PAD - N_NODES), (0, 0)))
    aT = a.T
    m2 = jnp.stack([a_2[0], W_mlp[0]], axis=1)
    b8 = jnp.pad(jnp.stack([jnp.zeros_like(b_mlp), b_mlp], axis=1),
                 ((0, 7), (0, 0)))
    a3T = aT[2 * D:2 * D + NRELA, :]

    src = jnp.concatenate([edge[0], edge_list_nhop[0]])
    dst = jnp.concatenate([edge[1], edge_list_nhop[1]])

    nblk = 512
    p1, p2, qa, qb, cvd = pl.pallas_call(
        _prep_nodes_body,
        grid=(N_PAD // nblk,),
        in_specs=[
            pl.BlockSpec((nblk, D), lambda i: (i, 0)),
            pl.BlockSpec((2 * D + NRELA, D), lambda i: (0, 0)),
            pl.BlockSpec((D, 2), lambda i: (0, 0)),
            pl.BlockSpec((8, 2), lambda i: (0, 0)),
        ],
        out_specs=[
            pl.BlockSpec((nblk, D), lambda i: (i, 0)),
            pl.BlockSpec((nblk, D), lambda i: (i, 0)),
            pl.BlockSpec((nblk, 2), lambda i: (i, 0)),
            pl.BlockSpec((nblk, 2), lambda i: (i, 0)),
            pl.BlockSpec((2, NRELA), lambda i: (0, 0)),
        ],
        out_shape=[
            jax.ShapeDtypeStruct((N_PAD, D), jnp.float32),
            jax.ShapeDtypeStruct((N_PAD, D), jnp.float32),
            jax.ShapeDtypeStruct((N_PAD, 2), jnp.float32),
            jax.ShapeDtypeStruct((N_PAD, 2), jnp.float32),
            jax.ShapeDtypeStruct((2, NRELA), jnp.float32),
        ],
    )(x, aT, m2, b8)

    zr, h2 = _sc_call(src, dst, edge_embed, edge_embed_nhop, qa, qb, p2, cvd)

    pblk = 512
    h = pl.pallas_call(
        _post_body,
        grid=(N_PAD // pblk,),
        in_specs=[
            pl.BlockSpec((pblk, 32), lambda i: (i, 0)),
            pl.BlockSpec((pblk, 32), lambda i: (i, 0)),
            pl.BlockSpec((pblk, D), lambda i: (i, 0)),
            pl.BlockSpec((pblk, D), lambda i: (i, 0)),
            pl.BlockSpec((pblk, D), lambda i: (i, 0)),
            pl.BlockSpec((NRELA, D), lambda i: (0, 0)),
        ],
        out_specs=pl.BlockSpec((pblk, D), lambda i: (i, 0)),
        out_shape=jax.ShapeDtypeStruct((N_PAD, D), jnp.float32),
    )(zr[:N_PAD], zr[N_PAD:], h2[:N_PAD], h2[N_PAD:], p1, a3T)

    return h[:N_NODES]


# speedup vs baseline: 4.5357x; 1.1036x over previous
import jax
import jax.numpy as jnp
from jax import lax
from jax.experimental import pallas as pl
from jax.experimental.pallas import tpu as pltpu
from jax.experimental.pallas import tpu_sc as plsc

N_NODES = 10000
N_PAD = 10240
D = 128
NRELA = 16
E1 = 256000
E2 = 64000
E_TOTAL = E1 + E2
NC = 2
NS = 16
K = 80
EPT = E_TOTAL // (NC * NS)
NCHUNK = EPT // K
STRIPE = N_PAD // NS


def _prep_nodes_body(x_ref, aT_ref, m_ref, b_ref,
                     p1_ref, p2_ref, qa_ref, qb_ref, cvd_ref):
    x = x_ref[...]
    p1 = jnp.dot(x, aT_ref[0:D, :], preferred_element_type=jnp.float32)
    p2 = jnp.dot(x, aT_ref[D:2 * D, :], preferred_element_type=jnp.float32)
    p1_ref[...] = p1
    p2_ref[...] = p2
    m = m_ref[...]
    qa_ref[...] = jnp.dot(p1, m, preferred_element_type=jnp.float32)
    qb_ref[...] = (jnp.dot(p2, m, preferred_element_type=jnp.float32)
                   + b_ref[0:1, :])
    cvd_ref[...] = lax.dot_general(
        m, aT_ref[2 * D:2 * D + NRELA, :],
        dimension_numbers=(((0,), (1,)), ((), ())),
        preferred_element_type=jnp.float32)


def _post_body(zr0_ref, zr1_ref, h20_ref, h21_ref, p1_ref, a3T_ref, out_ref):
    zr = zr0_ref[...] + zr1_ref[...]
    z = zr[:, 0:NRELA]
    rs = zr[:, NRELA:NRELA + 1]
    num = (h20_ref[...] + h21_ref[...] + p1_ref[...] * rs
           + jnp.dot(z, a3T_ref[...], preferred_element_type=jnp.float32))
    den = jnp.where(rs == 0.0, 1e-12, rs)
    h = num / den
    out_ref[...] = jnp.where(h > 0.0, h, jnp.exp(h) - 1.0)


def _lane_bcast(v, j):
    return lax.gather(
        v, jnp.full((16, 1), j, jnp.int32),
        dimension_numbers=lax.GatherDimensionNumbers(
            offset_dims=(), collapsed_slice_dims=(0,), start_index_map=(0,)),
        slice_sizes=(1,),
        mode=lax.GatherScatterMode.PROMISE_IN_BOUNDS)


def _sc_body(src_hbm, dst_hbm, ee1_hbm, ee2_hbm, qa_hbm, qb_hbm, p2_hbm,
             cvd_hbm,
             zr_out, h2_out,
             src_v0, dst_v0, ee_v0, src_v1, dst_v1, ee_v1,
             pay_v, prow_v, qa_v, qb_v, cvd_v,
             zr_sh, h2_sh, semL0, semL1, semG, semS):
    cid = lax.axis_index("c")
    sid = lax.axis_index("s")
    ebase = (cid * NS + sid) * EPT
    nb = sid * STRIPE
    iota16 = jnp.arange(16, dtype=jnp.int32)
    zeros16 = jnp.zeros((16,), jnp.int32)
    zero16f = jnp.zeros((16,), jnp.float32)
    onehot0 = jnp.where(iota16 == 0, 1.0, 0.0).astype(jnp.float32)

    pltpu.sync_copy(cvd_hbm, cvd_v)

    def _zrow(i, carry):
        for r in range(D // 16):
            prow_v[i, pl.ds(16 * r, 16)] = zero16f
        pay_v[i, pl.ds(0, 16)] = zero16f
        pay_v[i, pl.ds(16, 16)] = zero16f
        return carry

    lax.fori_loop(0, K, _zrow, 0)
    for t in range(STRIPE // K):
        pltpu.sync_copy(prow_v, h2_sh.at[pl.ds(nb + K * t, K)])
        pltpu.sync_copy(pay_v, zr_sh.at[pl.ds(nb + K * t, K)])
    plsc.subcore_barrier()

    cv3 = cvd_v[0, :]
    dv3 = cvd_v[1, :]
    cvb = [_lane_bcast(cv3, c) for c in range(NRELA)]
    dvb = [_lane_bcast(dv3, c) for c in range(NRELA)]

    inv_e = -1.0 / float(E_TOTAL)

    def lin_issue(cbase, sv, dv, ev, sem):
        pltpu.async_copy(src_hbm.at[pl.ds(cbase, K)], sv, sem)
        pltpu.async_copy(dst_hbm.at[pl.ds(cbase, K)], dv, sem)

        @pl.when(cbase < E1)
        def _():
            pltpu.async_copy(ee1_hbm.at[pl.ds(cbase, K)], ev, sem)

        @pl.when(cbase >= E1)
        def _():
            pltpu.async_copy(ee2_hbm.at[pl.ds(cbase - E1, K)], ev, sem)

    def lin_wait(sv, dv, ev, sem):
        pltpu.make_async_copy(src_hbm.at[pl.ds(0, K)], sv, sem).wait()
        pltpu.make_async_copy(dst_hbm.at[pl.ds(0, K)], dv, sem).wait()
        pltpu.make_async_copy(ee1_hbm.at[pl.ds(0, K)], ev, sem).wait()

    def ind_issue(sv, dv):
        pltpu.async_copy(qa_hbm.at[sv], qa_v, semG)
        pltpu.async_copy(qb_hbm.at[dv], qb_v, semG)
        pltpu.async_copy(p2_hbm.at[dv], prow_v, semG)

    def ind_wait(sv, dv):
        pltpu.make_async_copy(qa_hbm.at[sv], qa_v, semG).wait()
        pltpu.make_async_copy(qb_hbm.at[dv], qb_v, semG).wait()
        pltpu.make_async_copy(p2_hbm.at[dv], prow_v, semG).wait()

    def scat_issue(sv):
        pltpu.async_copy(pay_v, zr_sh.at[sv], semS, add=True)
        pltpu.async_copy(prow_v, h2_sh.at[sv], semS, add=True)

    def scat_wait(sv):
        pltpu.make_async_copy(pay_v, zr_sh.at[sv], semS).wait()
        pltpu.make_async_copy(prow_v, h2_sh.at[sv], semS).wait()

    def _compute(ev):
        for g in range(K // 16):
            rows = 16 * g + iota16
            q1g = plsc.load_gather(qa_v, [rows, zeros16])
            r1g = plsc.load_gather(qa_v, [rows, zeros16 + 1])
            q2g = plsc.load_gather(qb_v, [rows, zeros16])
            r2g = plsc.load_gather(qb_v, [rows, zeros16 + 1])
            col = plsc.load_gather(ev, [rows, zeros16])
            qe = col * cvb[0]
            re = col * dvb[0]
            for c in range(1, NRELA):
                col = plsc.load_gather(ev, [rows, zeros16 + c])
                qe = qe + col * cvb[c]
                re = re + col * dvb[c]
            s1 = q1g + q2g + qe
            s2 = r1g + r2g + re
            u = jnp.exp(-2.0 * jnp.abs(s2))
            th = jnp.sign(s2) * (1.0 - u) / (1.0 + u)
            lr = jnp.where(s1 > 0.0, s1, 0.2 * s1)
            ew = jnp.exp(lr * th * inv_e)
            for j in range(16):
                cs = _lane_bcast(ew, j)
                row = 16 * g + j
                for r in range(D // 16):
                    prow_v[row, pl.ds(16 * r, 16)] = (
                        prow_v[row, pl.ds(16 * r, 16)] * cs)
                pay_v[row, pl.ds(0, 16)] = ev[row, :] * cs
                pay_v[row, pl.ds(16, 16)] = cs * onehot0

    def _pair(k, carry):
        b0 = ebase + 2 * k * K
        lin_wait(src_v0, dst_v0, ee_v0, semL0)
        lin_issue(b0 + K, src_v1, dst_v1, ee_v1, semL1)
        ind_issue(src_v0, dst_v0)
        ind_wait(src_v0, dst_v0)
        _compute(ee_v0)
        scat_issue(src_v0)
        lin_wait(src_v1, dst_v1, ee_v1, semL1)
        scat_wait(src_v0)
        lin_issue(b0 + 2 * K, src_v0, dst_v0, ee_v0, semL0)
        ind_issue(src_v1, dst_v1)
        ind_wait(src_v1, dst_v1)
        _compute(ee_v1)
        scat_issue(src_v1)
        scat_wait(src_v1)
        return carry

    lin_issue(ebase, src_v0, dst_v0, ee_v0, semL0)
    lax.fori_loop(0, (NCHUNK - 1) // 2, _pair, 0)
    lin_wait(src_v0, dst_v0, ee_v0, semL0)
    ind_issue(src_v0, dst_v0)
    ind_wait(src_v0, dst_v0)
    _compute(ee_v0)
    scat_issue(src_v0)
    scat_wait(src_v0)
    plsc.subcore_barrier()

    ob = cid * N_PAD + nb
    pltpu.sync_copy(zr_sh.at[pl.ds(nb, STRIPE)], zr_out.at[pl.ds(ob, STRIPE)])
    pltpu.sync_copy(h2_sh.at[pl.ds(nb, STRIPE)], h2_out.at[pl.ds(ob, STRIPE)])


def _sc_call(src, dst, ee1, ee2, qa, qb, p2, cvd):
    f = pl.kernel(
        _sc_body,
        out_type=[jax.ShapeDtypeStruct((NC * N_PAD, 32), jnp.float32),
                  jax.ShapeDtypeStruct((NC * N_PAD, D), jnp.float32)],
        mesh=plsc.VectorSubcoreMesh(core_axis_name="c", subcore_axis_name="s"),
        compiler_params=pltpu.CompilerParams(
            needs_layout_passes=False, use_tc_tiling_on_sc=False),
        scratch_types=[
            pltpu.VMEM((K,), jnp.int32),
            pltpu.VMEM((K,), jnp.int32),
            pltpu.VMEM((K, NRELA), jnp.float32),
            pltpu.VMEM((K,), jnp.int32),
            pltpu.VMEM((K,), jnp.int32),
            pltpu.VMEM((K, NRELA), jnp.float32),
            pltpu.VMEM((K, 32), jnp.float32),
            pltpu.VMEM((K, D), jnp.float32),
            pltpu.VMEM((K, 2), jnp.float32),
            pltpu.VMEM((K, 2), jnp.float32),
            pltpu.VMEM((2, NRELA), jnp.float32),
            pltpu.VMEM_SHARED((N_PAD, 32), jnp.float32),
            pltpu.VMEM_SHARED((N_PAD, D), jnp.float32),
            pltpu.SemaphoreType.DMA,
            pltpu.SemaphoreType.DMA,
            pltpu.SemaphoreType.DMA,
            pltpu.SemaphoreType.DMA,
        ],
    )
    return f(src, dst, ee1, ee2, qa, qb, p2, cvd)


def kernel(input, edge, edge_embed, edge_list_nhop, edge_embed_nhop,
           a, a__2, a_2, W_mlp, b_mlp):
    x = jnp.pad(input, ((0, N_PAD - N_NODES), (0, 0)))
    aT = a.T
    m2 = jnp.stack([a_2[0], W_mlp[0]], axis=1)
    b8 = jnp.pad(jnp.stack([jnp.zeros_like(b_mlp), b_mlp], axis=1),
                 ((0, 7), (0, 0)))
    a3T = aT[2 * D:2 * D + NRELA, :]

    src = jnp.concatenate([edge[0], edge_list_nhop[0]])
    dst = jnp.concatenate([edge[1], edge_list_nhop[1]])

    nblk = 512
    p1, p2, qa, qb, cvd = pl.pallas_call(
        _prep_nodes_body,
        grid=(N_PAD // nblk,),
        in_specs=[
            pl.BlockSpec((nblk, D), lambda i: (i, 0)),
            pl.BlockSpec((2 * D + NRELA, D), lambda i: (0, 0)),
            pl.BlockSpec((D, 2), lambda i: (0, 0)),
            pl.BlockSpec((8, 2), lambda i: (0, 0)),
        ],
        out_specs=[
            pl.BlockSpec((nblk, D), lambda i: (i, 0)),
            pl.BlockSpec((nblk, D), lambda i: (i, 0)),
            pl.BlockSpec((nblk, 2), lambda i: (i, 0)),
            pl.BlockSpec((nblk, 2), lambda i: (i, 0)),
            pl.BlockSpec((2, NRELA), lambda i: (0, 0)),
        ],
        out_shape=[
            jax.ShapeDtypeStruct((N_PAD, D), jnp.float32),
            jax.ShapeDtypeStruct((N_PAD, D), jnp.float32),
            jax.ShapeDtypeStruct((N_PAD, 2), jnp.float32),
            jax.ShapeDtypeStruct((N_PAD, 2), jnp.float32),
            jax.ShapeDtypeStruct((2, NRELA), jnp.float32),
        ],
    )(x, aT, m2, b8)

    zr, h2 = _sc_call(src, dst, edge_embed, edge_embed_nhop, qa, qb, p2, cvd)

    pblk = 512
    h = pl.pallas_call(
        _post_body,
        grid=(N_PAD // pblk,),
        in_specs=[
            pl.BlockSpec((pblk, 32), lambda i: (i, 0)),
            pl.BlockSpec((pblk, 32), lambda i: (i, 0)),
            pl.BlockSpec((pblk, D), lambda i: (i, 0)),
            pl.BlockSpec((pblk, D), lambda i: (i, 0)),
            pl.BlockSpec((pblk, D), lambda i: (i, 0)),
            pl.BlockSpec((NRELA, D), lambda i: (0, 0)),
        ],
        out_specs=pl.BlockSpec((pblk, D), lambda i: (i, 0)),
        out_shape=jax.ShapeDtypeStruct((N_PAD, D), jnp.float32),
    )(zr[:N_PAD], zr[N_PAD:], h2[:N_PAD], h2[N_PAD:], p1, a3T)

    return h[:N_NODES]
